# baseline (speedup 1.0000x reference)
import functools

import jax
import jax.numpy as jnp
from jax import lax
from jax.experimental import pallas as pl
from jax.experimental.pallas import tpu as pltpu
from jax.experimental.pallas import tpu_sc as plsc

N = 10000
E = 320000
D = 128
EPS = 1e-5

NC = 2
NS = 16
NW = NC * NS
CH = 128
NCH = 79
EP = NW * NCH * CH
R = 10112
RPT = R // NS
DUMMY = N
EPT = NCH * CH
CHC = 64
EPT0 = 12544
EPT1 = 7680


def _dega_body(src_hbm, dst_hbm, ones_hbm, z_hbm,
               degp_out, de_out,
               src_v, dst_v, ones_v, degtab):
    c = lax.axis_index("c")
    s = lax.axis_index("s")
    wid = s * NC + c
    base = pl.multiple_of(s * RPT, 8)
    pltpu.sync_copy(src_hbm.at[wid], src_v)
    pltpu.sync_copy(dst_hbm.at[wid], dst_v)
    pltpu.sync_copy(ones_hbm, ones_v)
    pltpu.sync_copy(z_hbm.at[pl.ds(base, RPT)], degtab.at[pl.ds(base, RPT)])

    def _row(j, _):
        def _lane(i, _):
            st = pl.multiple_of(i * 16, 16)
            sv = src_v[j, pl.ds(st, 16)]
            dv = dst_v[j, pl.ds(st, 16)]
            dst_v[j, pl.ds(st, 16)] = jnp.where(sv == dv, DUMMY, dv)
            return 0
        return lax.fori_loop(0, CH // 16, _lane, 0)
    lax.fori_loop(0, NCH, _row, 0)

    plsc.subcore_barrier()

    def _chunk(j, _):
        pltpu.sync_copy(ones_v, degtab.at[dst_v.at[j]], add=True)
        return 0
    lax.fori_loop(0, NCH, _chunk, 0)

    plsc.subcore_barrier()

    pltpu.sync_copy(dst_v, de_out.at[wid])
    pltpu.sync_copy(degtab.at[pl.ds(base, RPT)], degp_out.at[c, pl.ds(base, RPT)])


def _scatter_body(g_hbm, src_hbm, de_hbm, z_hbm,
                  outp,
                  sidx, didx, gbuf, acc, sem0, sem1):
    c = lax.axis_index("c")
    s = lax.axis_index("s")
    base = pl.multiple_of(s * RPT, 8)
    pltpu.sync_copy(z_hbm.at[pl.ds(base, RPT)], acc.at[pl.ds(base, RPT)])

    def _run(ept, ebase):
        n_pair = ept // (2 * CHC)
        pltpu.sync_copy(src_hbm.at[pl.ds(ebase, ept)], sidx.at[pl.ds(0, ept)])
        pltpu.sync_copy(de_hbm.at[pl.ds(ebase, ept)], didx.at[pl.ds(0, ept)])
        plsc.subcore_barrier()
        pltpu.async_copy(g_hbm.at[sidx.at[pl.ds(0, CHC)]], gbuf.at[pl.ds(0, CHC)], sem0)

        def _pair(j2, _):
            e0 = pl.multiple_of(j2 * 2 * CHC, 8)
            e1 = pl.multiple_of(e0 + CHC, 8)
            en = pl.multiple_of(jnp.minimum(e0 + 2 * CHC, ept - 2 * CHC), 8)
            pltpu.async_copy(g_hbm.at[sidx.at[pl.ds(e1, CHC)]], gbuf.at[pl.ds(CHC, CHC)], sem1)
            pltpu.make_async_copy(g_hbm.at[sidx.at[pl.ds(0, CHC)]], gbuf.at[pl.ds(0, CHC)], sem0).wait()
            pltpu.sync_copy(gbuf.at[pl.ds(0, CHC)], acc.at[didx.at[pl.ds(e0, CHC)]], add=True)
            pltpu.async_copy(g_hbm.at[sidx.at[pl.ds(en, CHC)]], gbuf.at[pl.ds(0, CHC)], sem0)
            pltpu.make_async_copy(g_hbm.at[sidx.at[pl.ds(0, CHC)]], gbuf.at[pl.ds(CHC, CHC)], sem1).wait()
            pltpu.sync_copy(gbuf.at[pl.ds(CHC, CHC)], acc.at[didx.at[pl.ds(e1, CHC)]], add=True)
            return 0
        lax.fori_loop(0, n_pair, _pair, 0)
        pltpu.make_async_copy(g_hbm.at[sidx.at[pl.ds(0, CHC)]], gbuf.at[pl.ds(0, CHC)], sem0).wait()
        plsc.subcore_barrier()

    @pl.when(c == 0)
    def _():
        _run(EPT0, s * EPT0)

    @pl.when(c == 1)
    def _():
        _run(EPT1, 16 * EPT0 + s * EPT1)

    pltpu.sync_copy(acc.at[pl.ds(base, RPT)], outp.at[c, pl.ds(base, RPT)])


def _dense_body(x_ref, w0_ref, w1_ref, d0_ref, d1_ref,
                g_ref, self_ref, dinv_ref):
    x = x_ref[...]
    h = lax.dot_general(x, w0_ref[...], (((1,), (1,)), ((), ())),
                        preferred_element_type=jnp.float32)
    mean = jnp.mean(h, axis=0, keepdims=True)
    var = jnp.mean((h - mean) ** 2, axis=0, keepdims=True)
    h = jnp.maximum((h - mean) * lax.rsqrt(var + EPS), 0.0)
    h = lax.dot_general(h, w1_ref[...], (((1,), (1,)), ((), ())),
                        preferred_element_type=jnp.float32)
    deg = 1.0 + d0_ref[0:N, 0:1] + d1_ref[0:N, 0:1]
    dinv = lax.rsqrt(deg)
    g_ref[...] = h * dinv
    self_ref[...] = h / deg
    dinv_ref[...] = dinv


def _combine_body(p0_ref, p1_ref, self_ref, dinv_ref, o_ref):
    acc = p0_ref[0:N, :] + p1_ref[0:N, :]
    o_ref[...] = acc * dinv_ref[...] + self_ref[...]


def kernel(x, edge_index, W0, W1):
    ei = edge_index.astype(jnp.int32)
    src = ei[0]
    dst = ei[1]
    pad = EP - E
    src = jnp.concatenate([src, jnp.zeros((pad,), jnp.int32)]).reshape(NW, NCH, CH)
    dst = jnp.concatenate([dst, jnp.zeros((pad,), jnp.int32)]).reshape(NW, NCH, CH)

    ones128 = jnp.ones((CH, D), jnp.float32)
    z128 = jnp.zeros((R, D), jnp.float32)

    mesh = plsc.VectorSubcoreMesh(core_axis_name="c", subcore_axis_name="s")

    dega = pl.kernel(
        _dega_body,
        out_type=(
            jax.ShapeDtypeStruct((NC, R, D), jnp.float32),
            jax.ShapeDtypeStruct((NW, NCH, CH), jnp.int32),
        ),
        mesh=mesh,
        scratch_types=[
            pltpu.VMEM((NCH, CH), jnp.int32),
            pltpu.VMEM((NCH, CH), jnp.int32),
            pltpu.VMEM((CH, D), jnp.float32),
            pltpu.VMEM_SHARED((R, D), jnp.float32),
        ],
    )
    degp, dst_eff = dega(src, dst, ones128, z128)

    g, self_term, dinv = pl.pallas_call(
        _dense_body,
        out_shape=(
            jax.ShapeDtypeStruct((N, D), jnp.float32),
            jax.ShapeDtypeStruct((N, D), jnp.float32),
            jax.ShapeDtypeStruct((N, 1), jnp.float32),
        ),
    )(x, W0, W1, degp[0], degp[1])

    scat = pl.kernel(
        _scatter_body,
        out_type=jax.ShapeDtypeStruct((NC, R, D), jnp.float32),
        mesh=mesh,
        scratch_types=[
            pltpu.VMEM((EPT0,), jnp.int32),
            pltpu.VMEM((EPT0,), jnp.int32),
            pltpu.VMEM((2 * CHC, D), jnp.float32),
            pltpu.VMEM_SHARED((R, D), jnp.float32),
            pltpu.SemaphoreType.DMA,
            pltpu.SemaphoreType.DMA,
        ],
    )
    outp = scat(g, src.reshape(EP), dst_eff.reshape(EP), z128)

    out = pl.pallas_call(
        _combine_body,
        out_shape=jax.ShapeDtypeStruct((N, D), jnp.float32),
    )(outp[0], outp[1], self_term, dinv)
    return out


# speedup vs baseline: 1.0485x; 1.0485x over previous
import functools

import jax
import jax.numpy as jnp
from jax import lax
from jax.experimental import pallas as pl
from jax.experimental.pallas import tpu as pltpu
from jax.experimental.pallas import tpu_sc as plsc

N = 10000
E = 320000
D = 128
EPS = 1e-5

NC = 2
NS = 16
NW = NC * NS
CH = 128
NCH = 79
EP = NW * NCH * CH
R = 10112
RPT = R // NS
DUMMY = N
EPT = NCH * CH
CHC = 64
EPT0 = 14592
EPT1 = 5632


def _dega_body(src_hbm, dst_hbm, ones_hbm, z_hbm,
               degp_out, de_out,
               src_v, dst_v, ones_v, degtab):
    c = lax.axis_index("c")
    s = lax.axis_index("s")
    wid = s * NC + c
    base = pl.multiple_of(s * RPT, 8)
    pltpu.sync_copy(src_hbm.at[wid], src_v)
    pltpu.sync_copy(dst_hbm.at[wid], dst_v)
    pltpu.sync_copy(ones_hbm, ones_v)
    pltpu.sync_copy(z_hbm.at[pl.ds(base, RPT)], degtab.at[pl.ds(base, RPT)])

    def _row(j, _):
        def _lane(i, _):
            st = pl.multiple_of(i * 16, 16)
            sv = src_v[j, pl.ds(st, 16)]
            dv = dst_v[j, pl.ds(st, 16)]
            dst_v[j, pl.ds(st, 16)] = jnp.where(sv == dv, DUMMY, dv)
            return 0
        return lax.fori_loop(0, CH // 16, _lane, 0)
    lax.fori_loop(0, NCH, _row, 0)

    plsc.subcore_barrier()

    def _chunk(j, _):
        pltpu.sync_copy(ones_v, degtab.at[dst_v.at[j]], add=True)
        return 0
    lax.fori_loop(0, NCH, _chunk, 0)

    plsc.subcore_barrier()

    pltpu.sync_copy(dst_v, de_out.at[wid])
    pltpu.sync_copy(degtab.at[pl.ds(base, RPT)], degp_out.at[c, pl.ds(base, RPT)])


def _scatter_body(g_hbm, src_hbm, de_hbm, z_hbm,
                  outp,
                  sidx, didx, gbuf, acc, sem0, sem1):
    c = lax.axis_index("c")
    s = lax.axis_index("s")
    base = pl.multiple_of(s * RPT, 8)
    pltpu.sync_copy(z_hbm.at[pl.ds(base, RPT)], acc.at[pl.ds(base, RPT)])

    def _run(ept, ebase):
        n_pair = ept // (2 * CHC)
        pltpu.sync_copy(src_hbm.at[pl.ds(ebase, ept)], sidx.at[pl.ds(0, ept)])
        pltpu.sync_copy(de_hbm.at[pl.ds(ebase, ept)], didx.at[pl.ds(0, ept)])
        plsc.subcore_barrier()
        pltpu.async_copy(g_hbm.at[sidx.at[pl.ds(0, CHC)]], gbuf.at[pl.ds(0, CHC)], sem0)

        def _pair(j2, _):
            e0 = pl.multiple_of(j2 * 2 * CHC, 8)
            e1 = pl.multiple_of(e0 + CHC, 8)
            en = pl.multiple_of(jnp.minimum(e0 + 2 * CHC, ept - 2 * CHC), 8)
            pltpu.async_copy(g_hbm.at[sidx.at[pl.ds(e1, CHC)]], gbuf.at[pl.ds(CHC, CHC)], sem1)
            pltpu.make_async_copy(g_hbm.at[sidx.at[pl.ds(0, CHC)]], gbuf.at[pl.ds(0, CHC)], sem0).wait()
            pltpu.sync_copy(gbuf.at[pl.ds(0, CHC)], acc.at[didx.at[pl.ds(e0, CHC)]], add=True)
            pltpu.async_copy(g_hbm.at[sidx.at[pl.ds(en, CHC)]], gbuf.at[pl.ds(0, CHC)], sem0)
            pltpu.make_async_copy(g_hbm.at[sidx.at[pl.ds(0, CHC)]], gbuf.at[pl.ds(CHC, CHC)], sem1).wait()
            pltpu.sync_copy(gbuf.at[pl.ds(CHC, CHC)], acc.at[didx.at[pl.ds(e1, CHC)]], add=True)
            return 0
        lax.fori_loop(0, n_pair, _pair, 0)
        pltpu.make_async_copy(g_hbm.at[sidx.at[pl.ds(0, CHC)]], gbuf.at[pl.ds(0, CHC)], sem0).wait()
        plsc.subcore_barrier()

    @pl.when(c == 0)
    def _():
        _run(EPT0, s * EPT0)

    @pl.when(c == 1)
    def _():
        _run(EPT1, 16 * EPT0 + s * EPT1)

    pltpu.sync_copy(acc.at[pl.ds(base, RPT)], outp.at[c, pl.ds(base, RPT)])


def _dense_body(x_ref, w0_ref, w1_ref, d0_ref, d1_ref,
                g_ref, self_ref, dinv_ref):
    x = x_ref[...]
    h = lax.dot_general(x, w0_ref[...], (((1,), (1,)), ((), ())),
                        preferred_element_type=jnp.float32)
    mean = jnp.mean(h, axis=0, keepdims=True)
    var = jnp.mean((h - mean) ** 2, axis=0, keepdims=True)
    h = jnp.maximum((h - mean) * lax.rsqrt(var + EPS), 0.0)
    h = lax.dot_general(h, w1_ref[...], (((1,), (1,)), ((), ())),
                        preferred_element_type=jnp.float32)
    deg = 1.0 + d0_ref[0:N, 0:1] + d1_ref[0:N, 0:1]
    dinv = lax.rsqrt(deg)
    g_ref[...] = h * dinv
    self_ref[...] = h / deg
    dinv_ref[...] = dinv


def _combine_body(p0_ref, p1_ref, self_ref, dinv_ref, o_ref):
    acc = p0_ref[0:N, :] + p1_ref[0:N, :]
    o_ref[...] = acc * dinv_ref[...] + self_ref[...]


def kernel(x, edge_index, W0, W1):
    ei = edge_index.astype(jnp.int32)
    src = ei[0]
    dst = ei[1]
    pad = EP - E
    src = jnp.concatenate([src, jnp.zeros((pad,), jnp.int32)]).reshape(NW, NCH, CH)
    dst = jnp.concatenate([dst, jnp.zeros((pad,), jnp.int32)]).reshape(NW, NCH, CH)

    ones128 = jnp.ones((CH, D), jnp.float32)
    z128 = jnp.zeros((R, D), jnp.float32)

    mesh = plsc.VectorSubcoreMesh(core_axis_name="c", subcore_axis_name="s")

    dega = pl.kernel(
        _dega_body,
        out_type=(
            jax.ShapeDtypeStruct((NC, R, D), jnp.float32),
            jax.ShapeDtypeStruct((NW, NCH, CH), jnp.int32),
        ),
        mesh=mesh,
        scratch_types=[
            pltpu.VMEM((NCH, CH), jnp.int32),
            pltpu.VMEM((NCH, CH), jnp.int32),
            pltpu.VMEM((CH, D), jnp.float32),
            pltpu.VMEM_SHARED((R, D), jnp.float32),
        ],
    )
    degp, dst_eff = dega(src, dst, ones128, z128)

    g, self_term, dinv = pl.pallas_call(
        _dense_body,
        out_shape=(
            jax.ShapeDtypeStruct((N, D), jnp.float32),
            jax.ShapeDtypeStruct((N, D), jnp.float32),
            jax.ShapeDtypeStruct((N, 1), jnp.float32),
        ),
    )(x, W0, W1, degp[0], degp[1])

    scat = pl.kernel(
        _scatter_body,
        out_type=jax.ShapeDtypeStruct((NC, R, D), jnp.float32),
        mesh=mesh,
        scratch_types=[
            pltpu.VMEM((EPT0,), jnp.int32),
            pltpu.VMEM((EPT0,), jnp.int32),
            pltpu.VMEM((2 * CHC, D), jnp.float32),
            pltpu.VMEM_SHARED((R, D), jnp.float32),
            pltpu.SemaphoreType.DMA,
            pltpu.SemaphoreType.DMA,
        ],
    )
    outp = scat(g, src.reshape(EP), dst_eff.reshape(EP), z128)

    out = pl.pallas_call(
        _combine_body,
        out_shape=jax.ShapeDtypeStruct((N, D), jnp.float32),
    )(outp[0], outp[1], self_term, dinv)
    return out


# speedup vs baseline: 1.0687x; 1.0192x over previous
import functools

import jax
import jax.numpy as jnp
from jax import lax
from jax.experimental import pallas as pl
from jax.experimental.pallas import tpu as pltpu
from jax.experimental.pallas import tpu_sc as plsc

N = 10000
E = 320000
D = 128
EPS = 1e-5

NC = 2
NS = 16
NW = NC * NS
CH = 128
NCH = 79
EP = NW * NCH * CH
R = 10112
RPT = R // NS
DUMMY = N
EPT = NCH * CH
CHC = 64
EPT0 = 15616
EPT1 = 4608


def _dega_body(src_hbm, dst_hbm, ones_hbm, z_hbm,
               degp_out, de_out,
               src_v, dst_v, ones_v, degtab):
    c = lax.axis_index("c")
    s = lax.axis_index("s")
    wid = s * NC + c
    base = pl.multiple_of(s * RPT, 8)
    pltpu.sync_copy(src_hbm.at[wid], src_v)
    pltpu.sync_copy(dst_hbm.at[wid], dst_v)
    pltpu.sync_copy(ones_hbm, ones_v)
    pltpu.sync_copy(z_hbm.at[pl.ds(base, RPT)], degtab.at[pl.ds(base, RPT)])

    def _row(j, _):
        def _lane(i, _):
            st = pl.multiple_of(i * 16, 16)
            sv = src_v[j, pl.ds(st, 16)]
            dv = dst_v[j, pl.ds(st, 16)]
            dst_v[j, pl.ds(st, 16)] = jnp.where(sv == dv, DUMMY, dv)
            return 0
        return lax.fori_loop(0, CH // 16, _lane, 0)
    lax.fori_loop(0, NCH, _row, 0)

    plsc.subcore_barrier()

    def _chunk(j, _):
        pltpu.sync_copy(ones_v, degtab.at[dst_v.at[j]], add=True)
        return 0
    lax.fori_loop(0, NCH, _chunk, 0)

    plsc.subcore_barrier()

    pltpu.sync_copy(dst_v, de_out.at[wid])
    pltpu.sync_copy(degtab.at[pl.ds(base, RPT)], degp_out.at[c, pl.ds(base, RPT)])


def _scatter_body(g_hbm, src_hbm, de_hbm, z_hbm,
                  outp,
                  sidx, didx, gbuf, acc, sem0, sem1):
    c = lax.axis_index("c")
    s = lax.axis_index("s")
    base = pl.multiple_of(s * RPT, 8)
    pltpu.sync_copy(z_hbm.at[pl.ds(base, RPT)], acc.at[pl.ds(base, RPT)])

    def _run(ept, ebase):
        n_pair = ept // (2 * CHC)
        pltpu.sync_copy(src_hbm.at[pl.ds(ebase, ept)], sidx.at[pl.ds(0, ept)])
        pltpu.sync_copy(de_hbm.at[pl.ds(ebase, ept)], didx.at[pl.ds(0, ept)])
        plsc.subcore_barrier()
        pltpu.async_copy(g_hbm.at[sidx.at[pl.ds(0, CHC)]], gbuf.at[pl.ds(0, CHC)], sem0)

        def _pair(j2, _):
            e0 = pl.multiple_of(j2 * 2 * CHC, 8)
            e1 = pl.multiple_of(e0 + CHC, 8)
            en = pl.multiple_of(jnp.minimum(e0 + 2 * CHC, ept - 2 * CHC), 8)
            pltpu.async_copy(g_hbm.at[sidx.at[pl.ds(e1, CHC)]], gbuf.at[pl.ds(CHC, CHC)], sem1)
            pltpu.make_async_copy(g_hbm.at[sidx.at[pl.ds(0, CHC)]], gbuf.at[pl.ds(0, CHC)], sem0).wait()
            pltpu.sync_copy(gbuf.at[pl.ds(0, CHC)], acc.at[didx.at[pl.ds(e0, CHC)]], add=True)
            pltpu.async_copy(g_hbm.at[sidx.at[pl.ds(en, CHC)]], gbuf.at[pl.ds(0, CHC)], sem0)
            pltpu.make_async_copy(g_hbm.at[sidx.at[pl.ds(0, CHC)]], gbuf.at[pl.ds(CHC, CHC)], sem1).wait()
            pltpu.sync_copy(gbuf.at[pl.ds(CHC, CHC)], acc.at[didx.at[pl.ds(e1, CHC)]], add=True)
            return 0
        lax.fori_loop(0, n_pair, _pair, 0)
        pltpu.make_async_copy(g_hbm.at[sidx.at[pl.ds(0, CHC)]], gbuf.at[pl.ds(0, CHC)], sem0).wait()
        plsc.subcore_barrier()

    @pl.when(c == 0)
    def _():
        _run(EPT0, s * EPT0)

    @pl.when(c == 1)
    def _():
        _run(EPT1, 16 * EPT0 + s * EPT1)

    pltpu.sync_copy(acc.at[pl.ds(base, RPT)], outp.at[c, pl.ds(base, RPT)])


def _dense_body(x_ref, w0_ref, w1_ref, d0_ref, d1_ref,
                g_ref, self_ref, dinv_ref):
    x = x_ref[...]
    h = lax.dot_general(x, w0_ref[...], (((1,), (1,)), ((), ())),
                        preferred_element_type=jnp.float32)
    mean = jnp.mean(h, axis=0, keepdims=True)
    var = jnp.mean((h - mean) ** 2, axis=0, keepdims=True)
    h = jnp.maximum((h - mean) * lax.rsqrt(var + EPS), 0.0)
    h = lax.dot_general(h, w1_ref[...], (((1,), (1,)), ((), ())),
                        preferred_element_type=jnp.float32)
    deg = 1.0 + d0_ref[0:N, 0:1] + d1_ref[0:N, 0:1]
    dinv = lax.rsqrt(deg)
    g_ref[...] = h * dinv
    self_ref[...] = h / deg
    dinv_ref[...] = dinv


def _combine_body(p0_ref, p1_ref, self_ref, dinv_ref, o_ref):
    acc = p0_ref[0:N, :] + p1_ref[0:N, :]
    o_ref[...] = acc * dinv_ref[...] + self_ref[...]


def kernel(x, edge_index, W0, W1):
    ei = edge_index.astype(jnp.int32)
    src = ei[0]
    dst = ei[1]
    pad = EP - E
    src = jnp.concatenate([src, jnp.zeros((pad,), jnp.int32)]).reshape(NW, NCH, CH)
    dst = jnp.concatenate([dst, jnp.zeros((pad,), jnp.int32)]).reshape(NW, NCH, CH)

    ones128 = jnp.ones((CH, D), jnp.float32)
    z128 = jnp.zeros((R, D), jnp.float32)

    mesh = plsc.VectorSubcoreMesh(core_axis_name="c", subcore_axis_name="s")

    dega = pl.kernel(
        _dega_body,
        out_type=(
            jax.ShapeDtypeStruct((NC, R, D), jnp.float32),
            jax.ShapeDtypeStruct((NW, NCH, CH), jnp.int32),
        ),
        mesh=mesh,
        scratch_types=[
            pltpu.VMEM((NCH, CH), jnp.int32),
            pltpu.VMEM((NCH, CH), jnp.int32),
            pltpu.VMEM((CH, D), jnp.float32),
            pltpu.VMEM_SHARED((R, D), jnp.float32),
        ],
    )
    degp, dst_eff = dega(src, dst, ones128, z128)

    g, self_term, dinv = pl.pallas_call(
        _dense_body,
        out_shape=(
            jax.ShapeDtypeStruct((N, D), jnp.float32),
            jax.ShapeDtypeStruct((N, D), jnp.float32),
            jax.ShapeDtypeStruct((N, 1), jnp.float32),
        ),
    )(x, W0, W1, degp[0], degp[1])

    scat = pl.kernel(
        _scatter_body,
        out_type=jax.ShapeDtypeStruct((NC, R, D), jnp.float32),
        mesh=mesh,
        scratch_types=[
            pltpu.VMEM((EPT0,), jnp.int32),
            pltpu.VMEM((EPT0,), jnp.int32),
            pltpu.VMEM((2 * CHC, D), jnp.float32),
            pltpu.VMEM_SHARED((R, D), jnp.float32),
            pltpu.SemaphoreType.DMA,
            pltpu.SemaphoreType.DMA,
        ],
    )
    outp = scat(g, src.reshape(EP), dst_eff.reshape(EP), z128)

    out = pl.pallas_call(
        _combine_body,
        out_shape=jax.ShapeDtypeStruct((N, D), jnp.float32),
    )(outp[0], outp[1], self_term, dinv)
    return out


# speedup vs baseline: 1.0763x; 1.0071x over previous
import functools

import jax
import jax.numpy as jnp
from jax import lax
from jax.experimental import pallas as pl
from jax.experimental.pallas import tpu as pltpu
from jax.experimental.pallas import tpu_sc as plsc

N = 10000
E = 320000
D = 128
EPS = 1e-5

NC = 2
NS = 16
NW = NC * NS
CH = 128
NCH = 79
EP = NW * NCH * CH
R = 10112
RPT = R // NS
DUMMY = N
EPT = NCH * CH
CHC = 64
EPT0 = 16384
EPT1 = 3840


def _dega_body(src_hbm, dst_hbm, ones_hbm, z_hbm,
               degp_out, de_out,
               src_v, dst_v, ones_v, degtab):
    c = lax.axis_index("c")
    s = lax.axis_index("s")
    wid = s * NC + c
    base = pl.multiple_of(s * RPT, 8)
    pltpu.sync_copy(src_hbm.at[wid], src_v)
    pltpu.sync_copy(dst_hbm.at[wid], dst_v)
    pltpu.sync_copy(ones_hbm, ones_v)
    pltpu.sync_copy(z_hbm.at[pl.ds(base, RPT)], degtab.at[pl.ds(base, RPT)])

    def _row(j, _):
        def _lane(i, _):
            st = pl.multiple_of(i * 16, 16)
            sv = src_v[j, pl.ds(st, 16)]
            dv = dst_v[j, pl.ds(st, 16)]
            dst_v[j, pl.ds(st, 16)] = jnp.where(sv == dv, DUMMY, dv)
            return 0
        return lax.fori_loop(0, CH // 16, _lane, 0)
    lax.fori_loop(0, NCH, _row, 0)

    plsc.subcore_barrier()

    def _chunk(j, _):
        pltpu.sync_copy(ones_v, degtab.at[dst_v.at[j]], add=True)
        return 0
    lax.fori_loop(0, NCH, _chunk, 0)

    plsc.subcore_barrier()

    pltpu.sync_copy(dst_v, de_out.at[wid])
    pltpu.sync_copy(degtab.at[pl.ds(base, RPT)], degp_out.at[c, pl.ds(base, RPT)])


def _scatter_body(g_hbm, src_hbm, de_hbm, z_hbm,
                  outp,
                  sidx, didx, gbuf, acc, sem0, sem1):
    c = lax.axis_index("c")
    s = lax.axis_index("s")
    base = pl.multiple_of(s * RPT, 8)
    pltpu.sync_copy(z_hbm.at[pl.ds(base, RPT)], acc.at[pl.ds(base, RPT)])

    def _run(ept, ebase):
        n_pair = ept // (2 * CHC)
        pltpu.sync_copy(src_hbm.at[pl.ds(ebase, ept)], sidx.at[pl.ds(0, ept)])
        pltpu.sync_copy(de_hbm.at[pl.ds(ebase, ept)], didx.at[pl.ds(0, ept)])
        plsc.subcore_barrier()
        pltpu.async_copy(g_hbm.at[sidx.at[pl.ds(0, CHC)]], gbuf.at[pl.ds(0, CHC)], sem0)

        def _pair(j2, _):
            e0 = pl.multiple_of(j2 * 2 * CHC, 8)
            e1 = pl.multiple_of(e0 + CHC, 8)
            en = pl.multiple_of(jnp.minimum(e0 + 2 * CHC, ept - 2 * CHC), 8)
            pltpu.async_copy(g_hbm.at[sidx.at[pl.ds(e1, CHC)]], gbuf.at[pl.ds(CHC, CHC)], sem1)
            pltpu.make_async_copy(g_hbm.at[sidx.at[pl.ds(0, CHC)]], gbuf.at[pl.ds(0, CHC)], sem0).wait()
            pltpu.sync_copy(gbuf.at[pl.ds(0, CHC)], acc.at[didx.at[pl.ds(e0, CHC)]], add=True)
            pltpu.async_copy(g_hbm.at[sidx.at[pl.ds(en, CHC)]], gbuf.at[pl.ds(0, CHC)], sem0)
            pltpu.make_async_copy(g_hbm.at[sidx.at[pl.ds(0, CHC)]], gbuf.at[pl.ds(CHC, CHC)], sem1).wait()
            pltpu.sync_copy(gbuf.at[pl.ds(CHC, CHC)], acc.at[didx.at[pl.ds(e1, CHC)]], add=True)
            return 0
        lax.fori_loop(0, n_pair, _pair, 0)
        pltpu.make_async_copy(g_hbm.at[sidx.at[pl.ds(0, CHC)]], gbuf.at[pl.ds(0, CHC)], sem0).wait()
        plsc.subcore_barrier()

    @pl.when(c == 0)
    def _():
        _run(EPT0, s * EPT0)

    @pl.when(c == 1)
    def _():
        _run(EPT1, 16 * EPT0 + s * EPT1)

    pltpu.sync_copy(acc.at[pl.ds(base, RPT)], outp.at[c, pl.ds(base, RPT)])


def _dense_body(x_ref, w0_ref, w1_ref, d0_ref, d1_ref,
                g_ref, self_ref, dinv_ref):
    x = x_ref[...]
    h = lax.dot_general(x, w0_ref[...], (((1,), (1,)), ((), ())),
                        preferred_element_type=jnp.float32)
    mean = jnp.mean(h, axis=0, keepdims=True)
    var = jnp.mean((h - mean) ** 2, axis=0, keepdims=True)
    h = jnp.maximum((h - mean) * lax.rsqrt(var + EPS), 0.0)
    h = lax.dot_general(h, w1_ref[...], (((1,), (1,)), ((), ())),
                        preferred_element_type=jnp.float32)
    deg = 1.0 + d0_ref[0:N, 0:1] + d1_ref[0:N, 0:1]
    dinv = lax.rsqrt(deg)
    g_ref[...] = h * dinv
    self_ref[...] = h / deg
    dinv_ref[...] = dinv


def _combine_body(p0_ref, p1_ref, self_ref, dinv_ref, o_ref):
    acc = p0_ref[0:N, :] + p1_ref[0:N, :]
    o_ref[...] = acc * dinv_ref[...] + self_ref[...]


def kernel(x, edge_index, W0, W1):
    ei = edge_index.astype(jnp.int32)
    src = ei[0]
    dst = ei[1]
    pad = EP - E
    src = jnp.concatenate([src, jnp.zeros((pad,), jnp.int32)]).reshape(NW, NCH, CH)
    dst = jnp.concatenate([dst, jnp.zeros((pad,), jnp.int32)]).reshape(NW, NCH, CH)

    ones128 = jnp.ones((CH, D), jnp.float32)
    z128 = jnp.zeros((R, D), jnp.float32)

    mesh = plsc.VectorSubcoreMesh(core_axis_name="c", subcore_axis_name="s")

    dega = pl.kernel(
        _dega_body,
        out_type=(
            jax.ShapeDtypeStruct((NC, R, D), jnp.float32),
            jax.ShapeDtypeStruct((NW, NCH, CH), jnp.int32),
        ),
        mesh=mesh,
        scratch_types=[
            pltpu.VMEM((NCH, CH), jnp.int32),
            pltpu.VMEM((NCH, CH), jnp.int32),
            pltpu.VMEM((CH, D), jnp.float32),
            pltpu.VMEM_SHARED((R, D), jnp.float32),
        ],
    )
    degp, dst_eff = dega(src, dst, ones128, z128)

    g, self_term, dinv = pl.pallas_call(
        _dense_body,
        out_shape=(
            jax.ShapeDtypeStruct((N, D), jnp.float32),
            jax.ShapeDtypeStruct((N, D), jnp.float32),
            jax.ShapeDtypeStruct((N, 1), jnp.float32),
        ),
    )(x, W0, W1, degp[0], degp[1])

    scat = pl.kernel(
        _scatter_body,
        out_type=jax.ShapeDtypeStruct((NC, R, D), jnp.float32),
        mesh=mesh,
        scratch_types=[
            pltpu.VMEM((EPT0,), jnp.int32),
            pltpu.VMEM((EPT0,), jnp.int32),
            pltpu.VMEM((2 * CHC, D), jnp.float32),
            pltpu.VMEM_SHARED((R, D), jnp.float32),
            pltpu.SemaphoreType.DMA,
            pltpu.SemaphoreType.DMA,
        ],
    )
    outp = scat(g, src.reshape(EP), dst_eff.reshape(EP), z128)

    out = pl.pallas_call(
        _combine_body,
        out_shape=jax.ShapeDtypeStruct((N, D), jnp.float32),
    )(outp[0], outp[1], self_term, dinv)
    return out


# speedup vs baseline: 1.0776x; 1.0012x over previous
import jax
import jax.numpy as jnp
from jax import lax
from jax.experimental import pallas as pl
from jax.experimental.pallas import tpu as pltpu
from jax.experimental.pallas import tpu_sc as plsc

N = 10000
E = 320000
D = 128
EPS = 1e-5

NC = 2
NS = 16
NW = NC * NS
CH = 128
NCH = 79
EP = NW * NCH * CH
R = 10112
RPT = R // NS
DUMMY = N
CHC = 64
EPT0 = 16384
EPT1 = 3840


def _dega_body(src_hbm, dst_hbm, ones_hbm, z_hbm,
               degp_out, de_out,
               src_v, dst_v, ones_v, degtab):
    c = lax.axis_index("c")
    s = lax.axis_index("s")
    wid = s * NC + c
    base = pl.multiple_of(s * RPT, 8)
    pltpu.sync_copy(src_hbm.at[wid], src_v)
    pltpu.sync_copy(dst_hbm.at[wid], dst_v)
    pltpu.sync_copy(ones_hbm, ones_v)
    pltpu.sync_copy(z_hbm.at[pl.ds(base, RPT)], degtab.at[pl.ds(base, RPT)])

    def _row(j, _):
        def _lane(i, _):
            st = pl.multiple_of(i * 16, 16)
            sv = src_v[j, pl.ds(st, 16)]
            dv = dst_v[j, pl.ds(st, 16)]
            dst_v[j, pl.ds(st, 16)] = jnp.where(sv == dv, DUMMY, dv)
            return 0
        return lax.fori_loop(0, CH // 16, _lane, 0)
    lax.fori_loop(0, NCH, _row, 0)

    plsc.subcore_barrier()

    def _chunk(j, _):
        pltpu.sync_copy(ones_v, degtab.at[dst_v.at[j]], add=True)
        return 0
    lax.fori_loop(0, NCH, _chunk, 0)

    plsc.subcore_barrier()

    pltpu.sync_copy(dst_v, de_out.at[wid])
    pltpu.sync_copy(degtab.at[pl.ds(base, RPT)], degp_out.at[c, pl.ds(base, RPT)])


def _scatter_body(g_hbm, src_hbm, de_hbm, z_hbm,
                  outp,
                  sidx, didx, gbuf, acc, sem0, sem1):
    c = lax.axis_index("c")
    s = lax.axis_index("s")
    base = pl.multiple_of(s * RPT, 8)
    pltpu.sync_copy(z_hbm.at[pl.ds(base, RPT)], acc.at[pl.ds(base, RPT)])

    def _run(ept, ebase):
        n_pair = ept // (2 * CHC)
        pltpu.sync_copy(src_hbm.at[pl.ds(ebase, ept)], sidx.at[pl.ds(0, ept)])
        pltpu.sync_copy(de_hbm.at[pl.ds(ebase, ept)], didx.at[pl.ds(0, ept)])
        plsc.subcore_barrier()
        pltpu.async_copy(g_hbm.at[sidx.at[pl.ds(0, CHC)]], gbuf.at[pl.ds(0, CHC)], sem0)

        def _pair(j2, _):
            e0 = pl.multiple_of(j2 * 2 * CHC, 8)
            e1 = pl.multiple_of(e0 + CHC, 8)
            en = pl.multiple_of(jnp.minimum(e0 + 2 * CHC, ept - 2 * CHC), 8)
            pltpu.async_copy(g_hbm.at[sidx.at[pl.ds(e1, CHC)]], gbuf.at[pl.ds(CHC, CHC)], sem1)
            pltpu.make_async_copy(g_hbm.at[sidx.at[pl.ds(0, CHC)]], gbuf.at[pl.ds(0, CHC)], sem0).wait()
            pltpu.sync_copy(gbuf.at[pl.ds(0, CHC)], acc.at[didx.at[pl.ds(e0, CHC)]], add=True)
            pltpu.async_copy(g_hbm.at[sidx.at[pl.ds(en, CHC)]], gbuf.at[pl.ds(0, CHC)], sem0)
            pltpu.make_async_copy(g_hbm.at[sidx.at[pl.ds(0, CHC)]], gbuf.at[pl.ds(CHC, CHC)], sem1).wait()
            pltpu.sync_copy(gbuf.at[pl.ds(CHC, CHC)], acc.at[didx.at[pl.ds(e1, CHC)]], add=True)
            return 0
        lax.fori_loop(0, n_pair, _pair, 0)
        pltpu.make_async_copy(g_hbm.at[sidx.at[pl.ds(0, CHC)]], gbuf.at[pl.ds(0, CHC)], sem0).wait()
        plsc.subcore_barrier()

    @pl.when(c == 0)
    def _():
        _run(EPT0, s * EPT0)

    @pl.when(c == 1)
    def _():
        _run(EPT1, 16 * EPT0 + s * EPT1)

    pltpu.sync_copy(acc.at[pl.ds(base, RPT)], outp.at[c, pl.ds(base, RPT)])


def _dense_body(x_ref, w0_ref, w1_ref, d0_ref, d1_ref,
                g_ref, self_ref, dinv_ref):
    x = x_ref[...]
    h = lax.dot_general(x, w0_ref[...], (((1,), (1,)), ((), ())),
                        preferred_element_type=jnp.float32)
    mean = jnp.mean(h, axis=0, keepdims=True)
    var = jnp.mean((h - mean) ** 2, axis=0, keepdims=True)
    h = jnp.maximum((h - mean) * lax.rsqrt(var + EPS), 0.0)
    h = lax.dot_general(h, w1_ref[...], (((1,), (1,)), ((), ())),
                        preferred_element_type=jnp.float32)
    deg = 1.0 + d0_ref[0:N, 0:1] + d1_ref[0:N, 0:1]
    dinv = lax.rsqrt(deg)
    g_ref[...] = h * dinv
    self_ref[...] = h / deg
    dinv_ref[...] = dinv


def _combine_body(p0_ref, p1_ref, self_ref, dinv_ref, o_ref):
    acc = p0_ref[0:N, :] + p1_ref[0:N, :]
    o_ref[...] = acc * dinv_ref[...] + self_ref[...]


def kernel(x, edge_index, W0, W1):
    ei = edge_index.astype(jnp.int32)
    src = ei[0]
    dst = ei[1]
    pad = EP - E
    src = jnp.concatenate([src, jnp.zeros((pad,), jnp.int32)]).reshape(NW, NCH, CH)
    dst = jnp.concatenate([dst, jnp.zeros((pad,), jnp.int32)]).reshape(NW, NCH, CH)

    ones128 = jnp.ones((CH, D), jnp.float32)
    z128 = jnp.zeros((R, D), jnp.float32)

    mesh = plsc.VectorSubcoreMesh(core_axis_name="c", subcore_axis_name="s")

    dega = pl.kernel(
        _dega_body,
        out_type=(
            jax.ShapeDtypeStruct((NC, R, D), jnp.float32),
            jax.ShapeDtypeStruct((NW, NCH, CH), jnp.int32),
        ),
        mesh=mesh,
        scratch_types=[
            pltpu.VMEM((NCH, CH), jnp.int32),
            pltpu.VMEM((NCH, CH), jnp.int32),
            pltpu.VMEM((CH, D), jnp.float32),
            pltpu.VMEM_SHARED((R, D), jnp.float32),
        ],
    )
    degp, dst_eff = dega(src, dst, ones128, z128)

    g, self_term, dinv = pl.pallas_call(
        _dense_body,
        out_shape=(
            jax.ShapeDtypeStruct((N, D), jnp.float32),
            jax.ShapeDtypeStruct((N, D), jnp.float32),
            jax.ShapeDtypeStruct((N, 1), jnp.float32),
        ),
    )(x, W0, W1, degp[0], degp[1])

    scat = pl.kernel(
        _scatter_body,
        out_type=jax.ShapeDtypeStruct((NC, R, D), jnp.float32),
        mesh=mesh,
        scratch_types=[
            pltpu.VMEM((EPT0,), jnp.int32),
            pltpu.VMEM((EPT0,), jnp.int32),
            pltpu.VMEM((2 * CHC, D), jnp.float32),
            pltpu.VMEM_SHARED((R, D), jnp.float32),
            pltpu.SemaphoreType.DMA,
            pltpu.SemaphoreType.DMA,
        ],
    )
    outp = scat(g, src.reshape(EP), dst_eff.reshape(EP), z128)

    out = pl.pallas_call(
        _combine_body,
        out_shape=jax.ShapeDtypeStruct((N, D), jnp.float32),
    )(outp[0], outp[1], self_term, dinv)
    return out
